# baseline (speedup 1.0000x reference)
import jax
import jax.numpy as jnp
from jax.experimental import pallas as pl
from jax.experimental.pallas import tpu as pltpu

_N = 1_000_000
_LANES = 1024
_ROWS = 984
_BR = 24
_NBLK = _ROWS // _BR


def _threefry_bits(idx_u32):
    ks0 = jnp.uint32(0)
    ks1 = jnp.uint32(42)
    ks2 = ks0 ^ ks1 ^ jnp.uint32(0x1BD11BDA)
    ks = (ks0, ks1, ks2)
    r0 = (13, 15, 26, 6)
    r1 = (17, 29, 16, 24)

    x0 = jnp.broadcast_to(ks0, idx_u32.shape)
    x1 = idx_u32 + ks1

    def rounds(x0, x1, rots):
        for r in rots:
            x0 = x0 + x1
            x1 = (x1 << jnp.uint32(r)) | (x1 >> jnp.uint32(32 - r))
            x1 = x0 ^ x1
        return x0, x1

    for i, rots in enumerate((r0, r1, r0, r1, r0)):
        x0, x1 = rounds(x0, x1, rots)
        x0 = x0 + ks[(i + 1) % 3]
        x1 = x1 + ks[(i + 2) % 3] + jnp.uint32(i + 1)
    return x0 ^ x1


def _body(l_ref, out_ref, mval, midx):
    b = pl.program_id(0)

    @pl.when(b == 0)
    def _():
        mval[0] = -jnp.inf
        midx[0] = jnp.int32(0)

    base = b * (_BR * _LANES)
    row = jax.lax.broadcasted_iota(jnp.int32, (_BR, _LANES), 0)
    col = jax.lax.broadcasted_iota(jnp.int32, (_BR, _LANES), 1)
    gidx = base + row * _LANES + col

    bits = _threefry_bits(gidx.astype(jnp.uint32))
    fbits = (bits >> jnp.uint32(9)) | jnp.uint32(0x3F800000)
    f = jax.lax.bitcast_convert_type(fbits, jnp.float32) - jnp.float32(1.0)
    eps = jnp.float32(1e-10)
    u = jnp.maximum(eps, f * (jnp.float32(1.0) - eps) + eps)
    g = -jnp.log(-jnp.log(u))
    z = l_ref[...] + g

    bm = jnp.max(z)
    cand = jnp.where(z == bm, gidx, jnp.int32(0x7FFFFFFF))
    bi = jnp.min(cand)

    better = (bm > mval[0]) | ((bm == mval[0]) & (bi < midx[0]))

    @pl.when(better)
    def _():
        mval[0] = bm
        midx[0] = bi

    @pl.when(b == _NBLK - 1)
    def _():
        out_ref[0] = midx[0]


def kernel(logits):
    lp = jnp.pad(logits, (0, _ROWS * _LANES - _N),
                 constant_values=-jnp.inf).reshape(_ROWS, _LANES)
    out = pl.pallas_call(
        _body,
        grid=(_NBLK,),
        in_specs=[pl.BlockSpec((_BR, _LANES), lambda i: (i, 0))],
        out_specs=pl.BlockSpec(memory_space=pltpu.SMEM),
        out_shape=jax.ShapeDtypeStruct((1,), jnp.int32),
        scratch_shapes=[
            pltpu.SMEM((1,), jnp.float32),
            pltpu.SMEM((1,), jnp.int32),
        ],
    )(lp)
    return out[0]


# speedup vs baseline: 1.3736x; 1.3736x over previous
import jax
import jax.numpy as jnp
from jax.experimental import pallas as pl
from jax.experimental.pallas import tpu as pltpu

_N = 1_000_000
_LANES = 1024
_ROWS = 992
_BR = 32
_NBLK = _ROWS // _BR
_W = 256


def _threefry_bits(idx_u32):
    ks0 = jnp.uint32(0)
    ks1 = jnp.uint32(42)
    ks2 = ks0 ^ ks1 ^ jnp.uint32(0x1BD11BDA)
    ks = (ks0, ks1, ks2)
    r0 = (13, 15, 26, 6)
    r1 = (17, 29, 16, 24)

    x0 = jnp.broadcast_to(ks0, idx_u32.shape)
    x1 = idx_u32 + ks1

    def rounds(x0, x1, rots):
        for r in rots:
            x0 = x0 + x1
            x1 = (x1 << jnp.uint32(r)) | (x1 >> jnp.uint32(32 - r))
            x1 = x0 ^ x1
        return x0, x1

    for i, rots in enumerate((r0, r1, r0, r1, r0)):
        x0, x1 = rounds(x0, x1, rots)
        x0 = x0 + ks[(i + 1) % 3]
        x1 = x1 + ks[(i + 2) % 3] + jnp.uint32(i + 1)
    return x0 ^ x1


def _body(l_ref, out_ref, zmax, imax):
    b = pl.program_id(0)

    @pl.when(b == 0)
    def _():
        zmax[...] = jnp.full((8, _W), -jnp.inf, jnp.float32)
        imax[...] = jnp.zeros((8, _W), jnp.int32)

    row = jax.lax.broadcasted_iota(jnp.int32, (8, _W), 0)
    col = jax.lax.broadcasted_iota(jnp.int32, (8, _W), 1)
    rc = row * _LANES + col
    base = b * (_BR * _LANES)

    zm = zmax[...]
    im = imax[...]
    eps = jnp.float32(1e-10)
    for r8 in range(_BR // 8):
        for j in range(_LANES // _W):
            gidx = (base + r8 * (8 * _LANES) + j * _W) + rc
            bits = _threefry_bits(gidx.astype(jnp.uint32))
            fbits = (bits >> jnp.uint32(9)) | jnp.uint32(0x3F800000)
            f = jax.lax.bitcast_convert_type(fbits, jnp.float32)
            u = jnp.maximum(eps, (f - jnp.float32(1.0)) + eps)
            g = -jnp.log(-jnp.log(u))
            z = l_ref[pl.ds(r8 * 8, 8), pl.ds(j * _W, _W)] + g
            upd = z > zm
            zm = jnp.where(upd, z, zm)
            im = jnp.where(upd, gidx, im)
    zmax[...] = zm
    imax[...] = im

    @pl.when(b == _NBLK - 1)
    def _():
        m = jnp.max(zm)
        cand = jnp.where(zm == m, im, jnp.int32(0x7FFFFFFF))
        out_ref[0] = jnp.min(cand)


def kernel(logits):
    lp = jnp.pad(logits, (0, _ROWS * _LANES - _N),
                 constant_values=-jnp.inf).reshape(_ROWS, _LANES)
    out = pl.pallas_call(
        _body,
        grid=(_NBLK,),
        in_specs=[pl.BlockSpec((_BR, _LANES), lambda i: (i, 0))],
        out_specs=pl.BlockSpec(memory_space=pltpu.SMEM),
        out_shape=jax.ShapeDtypeStruct((1,), jnp.int32),
        scratch_shapes=[
            pltpu.VMEM((8, _W), jnp.float32),
            pltpu.VMEM((8, _W), jnp.int32),
        ],
    )(lp)
    return out[0]
